# baseline (speedup 1.0000x reference)
import functools

import jax
import jax.numpy as jnp
from jax import lax
from jax.experimental import pallas as pl
from jax.experimental.pallas import tpu as pltpu
from jax.experimental.pallas import tpu_sc as plsc

B = 16384
NUM = 13
NCAT = 26
VOCAB = 100000
D = 32
NT = 1 + NUM + NCAT
NW = 32
NB = 16
ROWS_PER_W = B // NW
NCHUNKS = ROWS_PER_W // NB
TPC = NB * NCAT


def _tok_body(xnum_hbm, gidx_hbm, sub_hbm, cdst_hbm, hdst_hbm, w_hbm, b_hbm,
              cls_hbm, table_hbm, out_hbm,
              gidx_v, sub_v, cdst_v, hdst_v, xnum_v, gbuf_v, catrow_v, head_v,
              w_v, b_v, cls_v, gsem, ssem):
    wid = lax.axis_index("s") * 2 + lax.axis_index("c")
    base = wid * ROWS_PER_W

    pltpu.sync_copy(w_hbm, w_v)
    pltpu.sync_copy(b_hbm, b_v)
    pltpu.sync_copy(cls_hbm, cls_v)
    cls0 = cls_v[pl.ds(0, 16)]
    cls1 = cls_v[pl.ds(16, 16)]
    iota16 = lax.iota(jnp.int32, 16)

    def chunk_body(c, carry):
        row0 = base + c * NB
        pltpu.sync_copy(gidx_hbm.at[pl.ds(row0 * NCAT, TPC)], gidx_v)
        pltpu.sync_copy(sub_hbm.at[pl.ds(row0 * NCAT, TPC)], sub_v)
        pltpu.sync_copy(cdst_hbm.at[pl.ds(row0 * NCAT, TPC)], cdst_v)
        pltpu.sync_copy(hdst_hbm.at[pl.ds(row0 * (1 + NUM), NB * (1 + NUM))],
                        hdst_v)
        pltpu.sync_copy(xnum_hbm.at[pl.ds(row0 * NUM, NB * NUM)], xnum_v)
        gcopy = pltpu.async_copy(table_hbm.at[gidx_v], gbuf_v, gsem)

        def row_body(i, carry2):
            head_v[i * (1 + NUM), pl.ds(0, 16)] = cls0
            head_v[i * (1 + NUM), pl.ds(16, 16)] = cls1
            for j in range(NUM):
                xij = plsc.load_gather(
                    xnum_v, [jnp.full((16,), i * NUM + j, jnp.int32)])
                for h in range(2):
                    off = (2 * j + h) * 16
                    head_v[i * (1 + NUM) + 1 + j, pl.ds(h * 16, 16)] = (
                        xij * w_v[pl.ds(off, 16)] + b_v[pl.ds(off, 16)])
            return carry2

        lax.fori_loop(0, NB, row_body, 0)
        gcopy.wait()

        def grp_body(g, carry2):
            rows = g * 16 + iota16
            sub = sub_v[pl.ds(g * 16, 16)]
            for d in range(D):
                val = plsc.load_gather(gbuf_v, [rows, sub + d])
                plsc.store_scatter(
                    catrow_v, [rows, jnp.full((16,), d, jnp.int32)], val)
            return carry2

        lax.fori_loop(0, TPC // 16, grp_body, 0)

        s1 = pltpu.async_copy(catrow_v, out_hbm.at[cdst_v], ssem)
        s2 = pltpu.async_copy(head_v, out_hbm.at[hdst_v], ssem)
        s1.wait()
        s2.wait()
        return carry

    lax.fori_loop(0, NCHUNKS, chunk_body, 0)


@functools.partial(
    pl.kernel,
    out_type=jax.ShapeDtypeStruct((B * NT, D), jnp.float32),
    mesh=plsc.VectorSubcoreMesh(core_axis_name="c", subcore_axis_name="s"),
    compiler_params=pltpu.CompilerParams(
        needs_layout_passes=False, use_tc_tiling_on_sc=False),
    scratch_types=[
        pltpu.VMEM((TPC,), jnp.int32),
        pltpu.VMEM((TPC,), jnp.int32),
        pltpu.VMEM((TPC,), jnp.int32),
        pltpu.VMEM((NB * (1 + NUM),), jnp.int32),
        pltpu.VMEM((NB * NUM,), jnp.float32),
        pltpu.VMEM((TPC, 128), jnp.float32),
        pltpu.VMEM((TPC, D), jnp.float32),
        pltpu.VMEM((NB * (1 + NUM), D), jnp.float32),
        pltpu.VMEM((NUM * D,), jnp.float32),
        pltpu.VMEM((NUM * D,), jnp.float32),
        pltpu.VMEM((D,), jnp.float32),
        pltpu.SemaphoreType.DMA,
        pltpu.SemaphoreType.DMA,
    ],
)
def _tok_kernel(*refs):
    _tok_body(*refs)


def kernel(x_num, x_cat, num_weights, num_bias, cat_tables, cls_token):
    flat = x_cat + (jnp.arange(NCAT, dtype=jnp.int32) * VOCAB)[None, :]
    gidx = (flat >> 2).reshape(-1)
    sub = ((flat & 3) << 5).reshape(-1)
    brow = jnp.arange(B, dtype=jnp.int32)[:, None] * NT
    cdst = (brow + (1 + NUM) + jnp.arange(NCAT, dtype=jnp.int32)[None, :])
    hdst = (brow + jnp.arange(1 + NUM, dtype=jnp.int32)[None, :])
    out = _tok_kernel(
        x_num.reshape(-1),
        gidx,
        sub,
        cdst.reshape(-1),
        hdst.reshape(-1),
        num_weights.reshape(-1),
        num_bias.reshape(-1),
        cls_token.reshape(-1),
        cat_tables.reshape(NCAT * VOCAB * D // 128, 128),
    )
    return out.reshape(B, NT, D)


# speedup vs baseline: 1.5840x; 1.5840x over previous
import functools

import jax
import jax.numpy as jnp
from jax import lax
from jax.experimental import pallas as pl
from jax.experimental.pallas import tpu as pltpu
from jax.experimental.pallas import tpu_sc as plsc

B = 16384
NUM = 13
NCAT = 26
VOCAB = 100000
D = 32
NT = 1 + NUM + NCAT
NW = 32
NB = 32
ROWS_PER_W = B // NW
NCHUNKS = ROWS_PER_W // NB
VCH = 4096


def _relayout_body(in_ref, out_ref):
    x = in_ref[0]
    t1 = x.T
    t2 = t1.reshape(VCH // 4, 4, D)
    out_ref[0] = jnp.concatenate([t2[:, q, :] for q in range(4)], axis=1)


def _relayout_table(tbl_t):
    nblk = (VOCAB + VCH - 1) // VCH
    rows_per_f = VOCAB * D // 128
    out = pl.pallas_call(
        _relayout_body,
        grid=(NCAT, nblk),
        in_specs=[pl.BlockSpec((1, D, VCH), lambda f, j: (f, 0, j))],
        out_specs=pl.BlockSpec((1, VCH // 4, 128), lambda f, j: (f, j, 0)),
        out_shape=jax.ShapeDtypeStruct((NCAT, rows_per_f, 128), jnp.float32),
    )(tbl_t)
    return out.reshape(NCAT * rows_per_f, 128)


def _post_body(in_ref, out_ref):
    y3 = in_ref[...].reshape(128, NT * D // 128, 128)
    parts = [
        y3[:, r, :].T.reshape(4, D, 128) for r in range(NT * D // 128)
    ]
    out_ref[...] = jnp.concatenate(parts, axis=0)


def _post_transpose(flat128):
    return pl.pallas_call(
        _post_body,
        grid=(B // 128,),
        in_specs=[pl.BlockSpec((NT * D // 128 * 128, 128), lambda j: (j, 0))],
        out_specs=pl.BlockSpec((NT, D, 128), lambda j: (0, 0, j)),
        out_shape=jax.ShapeDtypeStruct((NT, D, B), jnp.float32),
    )(flat128)


def _tok_body(xnum_hbm, idx_hbm, cdst_hbm, hdst_hbm, w_hbm, b_hbm, cls_hbm,
              table_hbm, out_hbm,
              idx_v, cdst_v, hdst_v, xnum_v, cat_v, head_v, w_v, b_v, cls_v,
              gsem, ssem):
    wid = lax.axis_index("s") * 2 + lax.axis_index("c")
    base = wid * ROWS_PER_W

    pltpu.sync_copy(w_hbm, w_v)
    pltpu.sync_copy(b_hbm, b_v)
    pltpu.sync_copy(cls_hbm, cls_v)
    cls0 = cls_v[pl.ds(0, 16)]
    cls1 = cls_v[pl.ds(16, 16)]

    def chunk_body(c, carry):
        row0 = base + c * NB
        pltpu.sync_copy(idx_hbm.at[pl.ds(row0 * NCAT, NB * NCAT)], idx_v)
        pltpu.sync_copy(cdst_hbm.at[pl.ds(row0 * NCAT, NB * NCAT)], cdst_v)
        pltpu.sync_copy(hdst_hbm.at[pl.ds(row0 * (1 + NUM), NB * (1 + NUM))],
                        hdst_v)
        pltpu.sync_copy(xnum_hbm.at[pl.ds(row0 * NUM, NB * NUM)], xnum_v)
        gcopy = pltpu.async_copy(table_hbm.at[idx_v], cat_v, gsem)

        def row_body(i, carry2):
            head_v[i * (1 + NUM), pl.ds(0, 16)] = cls0
            head_v[i * (1 + NUM), pl.ds(16, 16)] = cls1
            for j in range(NUM):
                xij = plsc.load_gather(
                    xnum_v, [jnp.full((16,), i * NUM + j, jnp.int32)])
                for h in range(2):
                    off = (2 * j + h) * 16
                    head_v[i * (1 + NUM) + 1 + j, pl.ds(h * 16, 16)] = (
                        xij * w_v[pl.ds(off, 16)] + b_v[pl.ds(off, 16)])
            return carry2

        lax.fori_loop(0, NB, row_body, 0)
        gcopy.wait()
        s1 = pltpu.async_copy(cat_v, out_hbm.at[cdst_v], ssem)
        s2 = pltpu.async_copy(head_v, out_hbm.at[hdst_v], ssem)
        s1.wait()
        s2.wait()
        return carry

    lax.fori_loop(0, NCHUNKS, chunk_body, 0)


@functools.partial(
    pl.kernel,
    out_type=jax.ShapeDtypeStruct((B * NT, D), jnp.float32),
    mesh=plsc.VectorSubcoreMesh(core_axis_name="c", subcore_axis_name="s"),
    compiler_params=pltpu.CompilerParams(
        needs_layout_passes=False, use_tc_tiling_on_sc=False),
    scratch_types=[
        pltpu.VMEM((NB * NCAT,), jnp.int32),
        pltpu.VMEM((NB * NCAT,), jnp.int32),
        pltpu.VMEM((NB * (1 + NUM),), jnp.int32),
        pltpu.VMEM((NB * NUM,), jnp.float32),
        pltpu.VMEM((NB * NCAT, D), jnp.float32),
        pltpu.VMEM((NB * (1 + NUM), D), jnp.float32),
        pltpu.VMEM((NUM * D,), jnp.float32),
        pltpu.VMEM((NUM * D,), jnp.float32),
        pltpu.VMEM((D,), jnp.float32),
        pltpu.SemaphoreType.DMA,
        pltpu.SemaphoreType.DMA,
    ],
)
def _tok_kernel(*refs):
    _tok_body(*refs)


def kernel(x_num, x_cat, num_weights, num_bias, cat_tables, cls_token):
    tbl128 = _relayout_table(jnp.swapaxes(cat_tables, 1, 2))
    idx = (x_cat + (jnp.arange(NCAT, dtype=jnp.int32) * VOCAB)[None, :])
    brow = jnp.arange(B, dtype=jnp.int32)[:, None] * NT
    cdst = (brow + (1 + NUM) + jnp.arange(NCAT, dtype=jnp.int32)[None, :])
    hdst = (brow + jnp.arange(1 + NUM, dtype=jnp.int32)[None, :])
    flat = _tok_kernel(
        x_num.reshape(-1),
        idx.reshape(-1),
        cdst.reshape(-1),
        hdst.reshape(-1),
        num_weights.reshape(-1),
        num_bias.reshape(-1),
        cls_token.reshape(-1),
        tbl128.reshape(NCAT * VOCAB, D),
    )
    out4 = _post_transpose(flat.reshape(B * NT * D // 128, 128))
    return jnp.transpose(out4, (2, 0, 1))


# speedup vs baseline: 1.9804x; 1.2503x over previous
import functools

import jax
import jax.numpy as jnp
from jax import lax
from jax.experimental import pallas as pl
from jax.experimental.pallas import tpu as pltpu
from jax.experimental.pallas import tpu_sc as plsc

B = 16384
NUM = 13
NCAT = 26
VOCAB = 100000
D = 32
NT = 1 + NUM + NCAT
NW = 32
NB = 32
ROWS_PER_W = B // NW
NCHUNKS = ROWS_PER_W // NB
VCH = 4096


def _relayout_body(in_ref, out_ref):
    x = in_ref[0]
    t1 = x.T
    for q in range(4):
        out_ref[0, :, pl.ds(D * q, D)] = t1[VCH // 4 * q:VCH // 4 * (q + 1), :]


NBLK = (VOCAB + VCH - 1) // VCH
ROWS_PER_F = NBLK * VCH // 4


def _relayout_table(tbl_t):
    out = pl.pallas_call(
        _relayout_body,
        grid=(NCAT, NBLK),
        in_specs=[pl.BlockSpec((1, D, VCH), lambda f, j: (f, 0, j))],
        out_specs=pl.BlockSpec((1, VCH // 4, 128), lambda f, j: (f, j, 0)),
        out_shape=jax.ShapeDtypeStruct((NCAT, ROWS_PER_F, 128), jnp.float32),
    )(tbl_t)
    return out.reshape(NCAT * ROWS_PER_F, 128)


def _post_body(in_ref, out_ref):
    y3 = in_ref[...].reshape(128, NT * D // 128, 128)
    parts = [
        y3[:, r, :].T.reshape(4, D, 128) for r in range(NT * D // 128)
    ]
    out_ref[...] = jnp.concatenate(parts, axis=0)


def _post_transpose(flat128):
    return pl.pallas_call(
        _post_body,
        grid=(B // 128,),
        in_specs=[pl.BlockSpec((NT * D // 128 * 128, 128), lambda j: (j, 0))],
        out_specs=pl.BlockSpec((NT, D, 128), lambda j: (0, 0, j)),
        out_shape=jax.ShapeDtypeStruct((NT, D, B), jnp.float32),
    )(flat128)


def _tok_body(xnum_hbm, idx_hbm, cdst_hbm, hdst_hbm, w_hbm, b_hbm, cls_hbm,
              table_hbm, out_hbm,
              idx_v, cdst_v, hdst_v, xnum_v, cat_v, head_v, w_v, b_v, cls_v,
              gsem, ssem):
    wid = lax.axis_index("s") * 2 + lax.axis_index("c")
    base = wid * ROWS_PER_W

    pltpu.sync_copy(w_hbm, w_v)
    pltpu.sync_copy(b_hbm, b_v)
    pltpu.sync_copy(cls_hbm, cls_v)
    cls0 = cls_v[pl.ds(0, 16)]
    cls1 = cls_v[pl.ds(16, 16)]

    def chunk_body(c, carry):
        row0 = base + c * NB
        pltpu.sync_copy(idx_hbm.at[pl.ds(row0 * NCAT, NB * NCAT)], idx_v)
        pltpu.sync_copy(cdst_hbm.at[pl.ds(row0 * NCAT, NB * NCAT)], cdst_v)
        pltpu.sync_copy(hdst_hbm.at[pl.ds(row0 * (1 + NUM), NB * (1 + NUM))],
                        hdst_v)
        pltpu.sync_copy(xnum_hbm.at[pl.ds(row0 * NUM, NB * NUM)], xnum_v)
        gcopy = pltpu.async_copy(table_hbm.at[idx_v], cat_v, gsem)

        def row_body(i, carry2):
            head_v[i * (1 + NUM), pl.ds(0, 16)] = cls0
            head_v[i * (1 + NUM), pl.ds(16, 16)] = cls1
            for j in range(NUM):
                xij = plsc.load_gather(
                    xnum_v, [jnp.full((16,), i * NUM + j, jnp.int32)])
                for h in range(2):
                    off = (2 * j + h) * 16
                    head_v[i * (1 + NUM) + 1 + j, pl.ds(h * 16, 16)] = (
                        xij * w_v[pl.ds(off, 16)] + b_v[pl.ds(off, 16)])
            return carry2

        lax.fori_loop(0, NB, row_body, 0)
        gcopy.wait()
        s1 = pltpu.async_copy(cat_v, out_hbm.at[cdst_v], ssem)
        s2 = pltpu.async_copy(head_v, out_hbm.at[hdst_v], ssem)
        s1.wait()
        s2.wait()
        return carry

    lax.fori_loop(0, NCHUNKS, chunk_body, 0)


@functools.partial(
    pl.kernel,
    out_type=jax.ShapeDtypeStruct((B * NT, D), jnp.float32),
    mesh=plsc.VectorSubcoreMesh(core_axis_name="c", subcore_axis_name="s"),
    compiler_params=pltpu.CompilerParams(
        needs_layout_passes=False, use_tc_tiling_on_sc=False),
    scratch_types=[
        pltpu.VMEM((NB * NCAT,), jnp.int32),
        pltpu.VMEM((NB * NCAT,), jnp.int32),
        pltpu.VMEM((NB * (1 + NUM),), jnp.int32),
        pltpu.VMEM((NB * NUM,), jnp.float32),
        pltpu.VMEM((NB * NCAT, D), jnp.float32),
        pltpu.VMEM((NB * (1 + NUM), D), jnp.float32),
        pltpu.VMEM((NUM * D,), jnp.float32),
        pltpu.VMEM((NUM * D,), jnp.float32),
        pltpu.VMEM((D,), jnp.float32),
        pltpu.SemaphoreType.DMA,
        pltpu.SemaphoreType.DMA,
    ],
)
def _tok_kernel(*refs):
    _tok_body(*refs)


def kernel(x_num, x_cat, num_weights, num_bias, cat_tables, cls_token):
    tbl128 = _relayout_table(jnp.swapaxes(cat_tables, 1, 2))
    v = x_cat
    w = v & (VCH - 1)
    idx = ((jnp.arange(NCAT, dtype=jnp.int32) * (4 * ROWS_PER_F))[None, :]
           + (v >> 12) * VCH + (w & (VCH // 4 - 1)) * 4 + (w >> 10))
    brow = jnp.arange(B, dtype=jnp.int32)[:, None] * NT
    cdst = (brow + (1 + NUM) + jnp.arange(NCAT, dtype=jnp.int32)[None, :])
    hdst = (brow + jnp.arange(1 + NUM, dtype=jnp.int32)[None, :])
    flat = _tok_kernel(
        x_num.reshape(-1),
        idx.reshape(-1),
        cdst.reshape(-1),
        hdst.reshape(-1),
        num_weights.reshape(-1),
        num_bias.reshape(-1),
        cls_token.reshape(-1),
        tbl128.reshape(NCAT * ROWS_PER_F * 4, D),
    )
    out4 = _post_transpose(flat.reshape(B * NT * D // 128, 128))
    return jnp.transpose(out4, (2, 0, 1))
